# baseline (speedup 1.0000x reference)
import functools

import jax
import jax.numpy as jnp
from jax.experimental import pallas as pl
from jax.experimental.pallas import tpu as pltpu


def _round_up(n, m):
    return ((n + m - 1) // m) * m


_LOG2E = 1.4426950408889634
_LN2 = 0.6931471805599453


def _softplus_base2(z):
    e = jnp.exp2(jnp.minimum(z, 126.0))
    return jnp.log2(1.0 + e) * _LN2


def _sep_dnn_kernel(
    x_ref, w1m_ref, b1m_ref, w1t_ref, b1t_ref,
    w2m_ref, b2m_ref, w2t_ref, b2t_ref, w3m_ref, b3m_ref,
    mu_ref, tau_ref,
    w1s_ref, w2ms_ref, w2ts_ref, w3ms_ref,
    *, h0, n_sub,
):
    @pl.when(pl.program_id(0) == 0)
    def _prep():
        w1s_ref[:, :h0] = w1m_ref[...].astype(jnp.bfloat16)
        w1s_ref[:, h0:] = (w1t_ref[...] * _LOG2E).astype(jnp.bfloat16)
        w2ms_ref[...] = w2m_ref[...].astype(jnp.bfloat16)
        w2ts_ref[...] = w2t_ref[...].astype(jnp.bfloat16)
        w3ms_ref[...] = w3m_ref[...].astype(jnp.bfloat16)

    sub = x_ref.shape[0] // n_sub
    b1m = b1m_ref[...]
    b1t2 = b1t_ref[...] * _LOG2E
    b2m = b2m_ref[...]
    b2t = b2t_ref[...]
    b3m = b3m_ref[...]
    for s in range(n_sub):
        rows = pl.ds(s * sub, sub)
        xb = x_ref[rows, :].astype(jnp.bfloat16)

        z1t = jnp.dot(xb, w1s_ref[:, h0:], preferred_element_type=jnp.float32)
        t = _softplus_base2(z1t + b1t2)
        z1m = jnp.dot(xb, w1s_ref[:, :h0], preferred_element_type=jnp.float32)
        h = jnp.maximum(z1m + b1m, 0.0)

        h1 = jnp.dot(h.astype(jnp.bfloat16), w2ms_ref[...],
                     preferred_element_type=jnp.float32)
        h1 = jnp.maximum(h1 + b2m, 0.0)
        mu = jnp.dot(h1.astype(jnp.bfloat16), w3ms_ref[...],
                     preferred_element_type=jnp.float32) + b3m
        tau = jnp.dot(t.astype(jnp.bfloat16), w2ts_ref[...],
                      preferred_element_type=jnp.float32) + b2t

        mu_ref[rows, :] = mu
        tau_ref[rows, :] = tau.astype(jnp.float32)


@functools.partial(jax.jit, static_argnames=("tile_b", "n_sub"))
def _sep_dnn_forward(x, w1m, b1m, w2m, b2m, w3m, b3m, w1t, b1t, w2t, b2t,
                     *, tile_b=2048, n_sub=8):
    B, D_in = x.shape
    D_out = w3m.shape[1]
    h0 = w1m.shape[1]
    h0t = w1t.shape[1]
    h1 = w2m.shape[1]

    tile_b = max(8, _round_up(min(tile_b, B), 8))
    B_pad = _round_up(B, tile_b)
    if B_pad != B:
        x = jnp.pad(x, ((0, B_pad - B), (0, 0)))
    grid = (B_pad // tile_b,)

    def full(arr):
        return pl.BlockSpec(arr.shape, lambda i: (0, 0))

    mu, tau = pl.pallas_call(
        functools.partial(_sep_dnn_kernel, h0=h0, n_sub=n_sub),
        out_shape=(
            jax.ShapeDtypeStruct((B_pad, D_out), jnp.float32),
            jax.ShapeDtypeStruct((B_pad, D_out), jnp.float32),
        ),
        grid=grid,
        in_specs=[
            pl.BlockSpec((tile_b, D_in), lambda i: (i, 0)),
            full(w1m), full(b1m), full(w1t), full(b1t),
            full(w2m), full(b2m), full(w2t), full(b2t),
            full(w3m), full(b3m),
        ],
        out_specs=(
            pl.BlockSpec((tile_b, D_out), lambda i: (i, 0)),
            pl.BlockSpec((tile_b, D_out), lambda i: (i, 0)),
        ),
        scratch_shapes=[
            pltpu.VMEM((D_in, h0 + h0t), jnp.bfloat16),
            pltpu.VMEM((h0, h1), jnp.bfloat16),
            pltpu.VMEM((h0t, D_out), jnp.bfloat16),
            pltpu.VMEM((h1, D_out), jnp.bfloat16),
        ],
        compiler_params=pltpu.CompilerParams(
            dimension_semantics=("arbitrary",),
        ),
    )(x, w1m, b1m, w1t, b1t, w2m, b2m, w2t, b2t, w3m, b3m)

    if B_pad != B:
        mu = mu[:B]
        tau = tau[:B]
    return mu, tau


def kernel(x, w1m, b1m, w2m, b2m, w3m, b3m, w1t, b1t, w2t, b2t):
    return _sep_dnn_forward(x, w1m, b1m, w2m, b2m, w3m, b3m,
                            w1t, b1t, w2t, b2t)


# speedup vs baseline: 1.0833x; 1.0833x over previous
import functools

import jax
import jax.numpy as jnp
from jax.experimental import pallas as pl
from jax.experimental.pallas import tpu as pltpu


def _round_up(n, m):
    return ((n + m - 1) // m) * m


_LOG2E = 1.4426950408889634
_LN2 = 0.6931471805599453


def _softplus_base2(z):
    e = jnp.exp2(jnp.minimum(z, 126.0))
    return jnp.log2(1.0 + e) * _LN2


def _sep_dnn_kernel(
    x_ref, w1m_ref, b1m_ref, w1t_ref, b1t_ref,
    w2m_ref, b2m_ref, w2t_ref, b2t_ref, w3m_ref, b3m_ref,
    mu_ref, tau_ref,
    w1s_ref, w2ms_ref, w2ts_ref, w3ms_ref,
    *, h0, n_sub,
):
    @pl.when(pl.program_id(0) == 0)
    def _prep():
        w1s_ref[:, :h0] = w1m_ref[...].astype(jnp.bfloat16)
        w1s_ref[:, h0:] = (w1t_ref[...] * _LOG2E).astype(jnp.bfloat16)
        w2ms_ref[...] = w2m_ref[...].astype(jnp.bfloat16)
        w2ts_ref[...] = w2t_ref[...].astype(jnp.bfloat16)
        w3ms_ref[...] = w3m_ref[...].astype(jnp.bfloat16)

    sub = x_ref.shape[0] // n_sub
    b1m = b1m_ref[...]
    b1t2 = b1t_ref[...] * _LOG2E
    b2m = b2m_ref[...]
    b2t = b2t_ref[...]
    b3m = b3m_ref[...]
    for s in range(n_sub):
        rows = pl.ds(s * sub, sub)
        xb = x_ref[rows, :].astype(jnp.bfloat16)

        z1m = jnp.dot(xb, w1s_ref[:, :h0], preferred_element_type=jnp.float32)
        h = jnp.maximum(z1m + b1m, 0.0)
        z1t = jnp.dot(xb, w1s_ref[:, h0:], preferred_element_type=jnp.float32)
        t = _softplus_base2(z1t + b1t2)

        h1 = jnp.dot(h.astype(jnp.bfloat16), w2ms_ref[...],
                     preferred_element_type=jnp.float32)
        h1 = jnp.maximum(h1 + b2m, 0.0)
        tau = jnp.dot(t.astype(jnp.bfloat16), w2ts_ref[...],
                      preferred_element_type=jnp.float32) + b2t

        mu = jnp.dot(h1.astype(jnp.bfloat16), w3ms_ref[...],
                     preferred_element_type=jnp.float32) + b3m

        mu_ref[rows, :] = mu
        tau_ref[rows, :] = tau.astype(jnp.float32)


@functools.partial(jax.jit, static_argnames=("tile_b", "n_sub"))
def _sep_dnn_forward(x, w1m, b1m, w2m, b2m, w3m, b3m, w1t, b1t, w2t, b2t,
                     *, tile_b=2048, n_sub=8):
    B, D_in = x.shape
    D_out = w3m.shape[1]
    h0 = w1m.shape[1]
    h0t = w1t.shape[1]
    h1 = w2m.shape[1]

    tile_b = max(8, _round_up(min(tile_b, B), 8))
    B_pad = _round_up(B, tile_b)
    if B_pad != B:
        x = jnp.pad(x, ((0, B_pad - B), (0, 0)))
    grid = (B_pad // tile_b,)

    def full(arr):
        return pl.BlockSpec(arr.shape, lambda i: (0, 0))

    mu, tau = pl.pallas_call(
        functools.partial(_sep_dnn_kernel, h0=h0, n_sub=n_sub),
        out_shape=(
            jax.ShapeDtypeStruct((B_pad, D_out), jnp.float32),
            jax.ShapeDtypeStruct((B_pad, D_out), jnp.float32),
        ),
        grid=grid,
        in_specs=[
            pl.BlockSpec((tile_b, D_in), lambda i: (i, 0)),
            full(w1m), full(b1m), full(w1t), full(b1t),
            full(w2m), full(b2m), full(w2t), full(b2t),
            full(w3m), full(b3m),
        ],
        out_specs=(
            pl.BlockSpec((tile_b, D_out), lambda i: (i, 0)),
            pl.BlockSpec((tile_b, D_out), lambda i: (i, 0)),
        ),
        scratch_shapes=[
            pltpu.VMEM((D_in, h0 + h0t), jnp.bfloat16),
            pltpu.VMEM((h0, h1), jnp.bfloat16),
            pltpu.VMEM((h0t, D_out), jnp.bfloat16),
            pltpu.VMEM((h1, D_out), jnp.bfloat16),
        ],
        compiler_params=pltpu.CompilerParams(
            dimension_semantics=("arbitrary",),
        ),
    )(x, w1m, b1m, w1t, b1t, w2m, b2m, w2t, b2t, w3m, b3m)

    if B_pad != B:
        mu = mu[:B]
        tau = tau[:B]
    return mu, tau


def kernel(x, w1m, b1m, w2m, b2m, w3m, b3m, w1t, b1t, w2t, b2t):
    return _sep_dnn_forward(x, w1m, b1m, w2m, b2m, w3m, b3m,
                            w1t, b1t, w2t, b2t)
